# baseline (speedup 1.0000x reference)
import functools

import jax
import jax.numpy as jnp
from jax import lax
from jax.experimental import pallas as pl
from jax.experimental.pallas import tpu as pltpu, tpu_sc as plsc

_NX, _NY, _NZ = 432, 496, 1
_B = 4
_CF = 64
_GRID = _NZ * _NX * _NY
_RTOT = _B * _GRID
_YB = 16
_SBLK = _YB * _NX
_HBLK = _SBLK // 2
_NYB = _NY // _YB
_NBLK = _B * _NYB
_C64_ROWS = 2 * (_RTOT + _SBLK) // 2
_ACAN = _RTOT + _SBLK
_NW = 32
_ROWN = _RTOT // _NW
_CAP = ((_ROWN + 127) // 128) * 128
_DCH = 128


def _pass_a(flat, pillar, a0, a1, a2):
    p = flat.shape[0]
    scch = 8000
    assert p % scch == 0 and p % 16 == 0
    nch_scan = p // scch
    mesh = plsc.VectorSubcoreMesh(core_axis_name="c", subcore_axis_name="s")

    @functools.partial(
        pl.kernel,
        out_type=[
            jax.ShapeDtypeStruct((_C64_ROWS, _CF), jnp.float32),
            jax.ShapeDtypeStruct((_ACAN,), jnp.float32),
            jax.ShapeDtypeStruct((_ACAN,), jnp.float32),
            jax.ShapeDtypeStruct((_ACAN,), jnp.float32),
            jax.ShapeDtypeStruct((_RTOT,), jnp.int32),
        ],
        mesh=mesh,
        compiler_params=pltpu.CompilerParams(
            needs_layout_passes=False, use_tc_tiling_on_sc=False),
        scratch_types=[
            pltpu.VMEM((_ROWN,), jnp.int32),
            pltpu.VMEM((_CAP,), jnp.int32),
            pltpu.VMEM((_CAP,), jnp.int32),
            pltpu.VMEM((scch,), jnp.int32),
            pltpu.VMEM((scch,), jnp.int32),
            pltpu.VMEM((_DCH,), jnp.int32),
            pltpu.VMEM((_DCH,), jnp.int32),
            pltpu.VMEM((_DCH,), jnp.int32),
            pltpu.VMEM((_DCH, _CF), jnp.float32),
            pltpu.VMEM((_DCH,), jnp.float32),
            pltpu.VMEM((_DCH,), jnp.float32),
            pltpu.VMEM((_DCH,), jnp.float32),
            pltpu.SemaphoreType.DMA,
            pltpu.SemaphoreType.DMA,
        ],
    )
    def k(flat_hbm, pil_hbm, a0_hbm, a1_hbm, a2_hbm,
          canvas_hbm, c0_hbm, c1_hbm, c2_hbm, occ_hbm,
          map_v, rows_v, winners_v, idx0, idx1, cchunk, wchunk, lchunk,
          rowbuf, ab0, ab1, ab2, gsem, ssem):
        nc = 2
        wid = lax.axis_index("s") * nc + lax.axis_index("c")
        base = wid * _ROWN
        iota = lax.iota(jnp.int32, 16)
        zeros16 = jnp.zeros((16,), jnp.int32)

        def p0(i, _):
            map_v[pl.ds(i * 16, 16)] = zeros16
            return 0
        lax.fori_loop(0, _ROWN // 16, p0, 0)

        def p1(i, _):
            g = i * 16 + iota
            rows_v[pl.ds(i * 16, 16)] = _RTOT + (g & 127)
            winners_v[pl.ds(i * 16, 16)] = zeros16
            return 0
        lax.fori_loop(0, _CAP // 16, p1, 0)

        def scan_chunk(buf, c):
            def p2(i, _):
                idx = buf[pl.ds(i * 16, 16)]
                m = (idx >= base) & (idx < base + _ROWN)
                ids = c * scch + i * 16 + 1 + iota
                plsc.store_scatter(map_v, [idx - base], ids, mask=m)
                return 0
            lax.fori_loop(0, scch // 16, p2, 0)

        bufs = [idx0, idx1]
        for c in range(nch_scan):
            buf = bufs[c % 2]
            pltpu.sync_copy(flat_hbm.at[pl.ds(c * scch, scch)], buf)
            scan_chunk(buf, c)

        def p3(g, n):
            v = map_v[pl.ds(g * 16, 16)]
            m = v > 0
            cb = base + g * 16
            bc = cb // _SBLK
            w = cb - bc * _SBLK
            h = w // _HBLK
            off = w - h * _HBLK
            r64 = 2 * (bc * _HBLK + off) + h + 2 * iota
            plsc.store_compressed(rows_v.at[pl.ds(n, 16)], r64, mask=m)
            plsc.store_compressed(winners_v.at[pl.ds(n, 16)], v - 1, mask=m)
            return n + jnp.sum(m.astype(jnp.int32))
        n = lax.fori_loop(0, _ROWN // 16, p3, jnp.int32(0))

        def p4(j, _):
            for t in range(_DCH // 16):
                o = pl.ds(t * 16, 16)
                r64 = rows_v[pl.ds(j * _DCH + t * 16, 16)]
                cchunk[o] = r64
                wchunk[o] = winners_v[pl.ds(j * _DCH + t * 16, 16)]
                r = r64 >> 1
                bc = ((r >> 7) * 9710) >> 18
                lchunk[o] = bc * _SBLK + (r64 & 1) * _HBLK + (r - bc * _HBLK)
            d0 = pltpu.async_copy(pil_hbm.at[wchunk], rowbuf, gsem)
            d1 = pltpu.async_copy(a0_hbm.at[wchunk], ab0, gsem)
            d2 = pltpu.async_copy(a1_hbm.at[wchunk], ab1, gsem)
            d3 = pltpu.async_copy(a2_hbm.at[wchunk], ab2, gsem)
            d0.wait(); d1.wait(); d2.wait(); d3.wait()
            s0 = pltpu.async_copy(rowbuf, canvas_hbm.at[cchunk], ssem)
            s1 = pltpu.async_copy(ab0, c0_hbm.at[lchunk], ssem)
            s2 = pltpu.async_copy(ab1, c1_hbm.at[lchunk], ssem)
            s3 = pltpu.async_copy(ab2, c2_hbm.at[lchunk], ssem)
            s0.wait(); s1.wait(); s2.wait(); s3.wait()
            return 0
        lax.fori_loop(0, (n + _DCH - 1) // _DCH, p4, 0)

        pltpu.sync_copy(map_v, occ_hbm.at[pl.ds(base, _ROWN)])

    return k(flat, pillar, a0, a1, a2)


def _pass_b(canvas, occ, adds):
    def body(cv_ref, occ_ref, a0_ref, a1_ref, a2_ref, o1_ref, o2_ref):
        b = pl.program_id(0)
        rowmask = lax.broadcasted_iota(jnp.int32, (_B, 1), 0) == b
        occv = jnp.sum(jnp.where(rowmask, occ_ref[...], 0), axis=0)
        occm = (occv > 0)[None, :]
        t = jnp.transpose(cv_ref[...])
        t1 = jnp.concatenate([t[:_CF], t[_CF:]], axis=1)
        t1 = jnp.where(occm, t1, 0.0)
        fr = jnp.float32(0.0)
        rows = [jnp.sum(jnp.where(rowmask, r[...], fr), axis=0)
                for r in (a0_ref, a1_ref, a2_ref)]
        t2 = jnp.where(occm, jnp.stack(rows, axis=0), 0.0)
        for yy in range(_YB):
            lo, hi = yy * _NX, (yy + 1) * _NX
            o1_ref[0, :, yy, :] = t1[:, lo:hi]
            o2_ref[0, :, yy, :] = t2[:, lo:hi]

    return pl.pallas_call(
        body,
        grid=(_B, _NYB),
        in_specs=[
            pl.BlockSpec((_HBLK, 2 * _CF), lambda b, y: (b * _NYB + y, 0)),
            pl.BlockSpec((_B, _SBLK), lambda b, y: (0, y)),
            pl.BlockSpec((_B, _SBLK), lambda b, y: (0, y)),
            pl.BlockSpec((_B, _SBLK), lambda b, y: (0, y)),
            pl.BlockSpec((_B, _SBLK), lambda b, y: (0, y)),
        ],
        out_specs=[
            pl.BlockSpec((1, _CF, _YB, _NX), lambda b, y: (b, 0, y, 0)),
            pl.BlockSpec((1, 3, _YB, _NX), lambda b, y: (b, 0, y, 0)),
        ],
        out_shape=[
            jax.ShapeDtypeStruct((_B, _CF, _NY, _NX), jnp.float32),
            jax.ShapeDtypeStruct((_B, 3, _NY, _NX), jnp.float32),
        ],
    )(canvas, occ, *adds)


def kernel(add_features_to_map, pillar_features, voxel_coords):
    vc = voxel_coords.astype(jnp.int32)
    flat = vc[:, 0] * _GRID + vc[:, 1] + vc[:, 2] * _NX + vc[:, 3]
    a0, a1, a2 = (add_features_to_map[:, ch] for ch in range(3))

    canvas, c0, c1, c2, occ = _pass_a(flat, pillar_features, a0, a1, a2)
    adds = [c[:_RTOT].reshape(_B, _GRID) for c in (c0, c1, c2)]
    return _pass_b(canvas.reshape(_C64_ROWS // 2, 2 * _CF),
                   occ.reshape(_B, _GRID), adds)


# speedup vs baseline: 1.0827x; 1.0827x over previous
import functools

import jax
import jax.numpy as jnp
from jax import lax
from jax.experimental import pallas as pl
from jax.experimental.pallas import tpu as pltpu, tpu_sc as plsc

_NX, _NY, _NZ = 432, 496, 1
_B = 4
_CF = 64
_GRID = _NZ * _NX * _NY
_RTOT = _B * _GRID
_YB = 16
_SBLK = _YB * _NX
_NYB = _NY // _YB
_CW = 128
_CAN_ROWS = _RTOT + _SBLK
_NW = 32
_ROWN = _RTOT // _NW
_CAP = ((_ROWN + 127) // 128) * 128 + 128
_DCH = 128


def _pass_a(flat, src):
    p = flat.shape[0]
    scch = 8000
    assert p % scch == 0 and p % 16 == 0
    nch_scan = p // scch
    mesh = plsc.VectorSubcoreMesh(core_axis_name="c", subcore_axis_name="s")

    @functools.partial(
        pl.kernel,
        out_type=[
            jax.ShapeDtypeStruct((_CAN_ROWS, _CW), jnp.float32),
            jax.ShapeDtypeStruct((_RTOT,), jnp.int32),
        ],
        mesh=mesh,
        compiler_params=pltpu.CompilerParams(needs_layout_passes=False),
        scratch_types=[
            pltpu.VMEM((_ROWN,), jnp.int32),
            pltpu.VMEM((_CAP,), jnp.int32),
            pltpu.VMEM((_CAP,), jnp.int32),
            pltpu.VMEM((scch,), jnp.int32),
            pltpu.VMEM((scch,), jnp.int32),
            pltpu.VMEM((_DCH,), jnp.int32),
            pltpu.VMEM((_DCH,), jnp.int32),
            pltpu.VMEM((_DCH,), jnp.int32),
            pltpu.VMEM((_DCH,), jnp.int32),
            pltpu.VMEM((_DCH, _CW), jnp.float32),
            pltpu.VMEM((_DCH, _CW), jnp.float32),
            pltpu.SemaphoreType.DMA,
            pltpu.SemaphoreType.DMA,
        ],
    )
    def k(flat_hbm, src_hbm, canvas_hbm, occ_hbm,
          map_v, cells_v, winners_v, idx0, idx1, cchunk0, wchunk0,
          cchunk1, wchunk1, rowbuf0, rowbuf1, gsem, ssem):
        nc = 2
        wid = lax.axis_index("s") * nc + lax.axis_index("c")
        base = wid * _ROWN
        iota = lax.iota(jnp.int32, 16)
        zeros16 = jnp.zeros((16,), jnp.int32)

        def p0(i, _):
            map_v[pl.ds(i * 16, 16)] = zeros16
            return 0
        lax.fori_loop(0, _ROWN // 16, p0, 0)

        def p1(i, _):
            g = i * 16 + iota
            cells_v[pl.ds(i * 16, 16)] = _RTOT + (g & 127)
            winners_v[pl.ds(i * 16, 16)] = zeros16
            return 0
        lax.fori_loop(0, _CAP // 16, p1, 0)

        def scan_chunk(buf, c):
            def p2(i, _):
                idx = buf[pl.ds(i * 16, 16)]
                m = (idx >= base) & (idx < base + _ROWN)
                ids = c * scch + i * 16 + 1 + iota
                plsc.store_scatter(map_v, [idx - base], ids, mask=m)
                return 0
            lax.fori_loop(0, scch // 16, p2, 0)

        bufs = [idx0, idx1]
        pltpu.async_copy(flat_hbm.at[pl.ds(0, scch)], bufs[0], gsem)
        for c in range(nch_scan):
            buf = bufs[c % 2]
            pltpu.make_async_copy(
                flat_hbm.at[pl.ds(c * scch, scch)], buf, gsem).wait()
            if c + 1 < nch_scan:
                pltpu.async_copy(flat_hbm.at[pl.ds((c + 1) * scch, scch)],
                                 bufs[(c + 1) % 2], gsem)
            scan_chunk(buf, c)

        def p3(g, n):
            v = map_v[pl.ds(g * 16, 16)]
            m = v > 0
            plsc.store_compressed(cells_v.at[pl.ds(n, 16)],
                                  base + g * 16 + iota, mask=m)
            plsc.store_compressed(winners_v.at[pl.ds(n, 16)], v - 1, mask=m)
            return n + jnp.sum(m.astype(jnp.int32))
        n = lax.fori_loop(0, _ROWN // 16, p3, jnp.int32(0))

        cbufs = [(cchunk0, wchunk0, rowbuf0), (cchunk1, wchunk1, rowbuf1)]

        def stage(j, cc, wc):
            for t in range(_DCH // 16):
                o = pl.ds(t * 16, 16)
                cc[o] = cells_v[pl.ds(j * _DCH + t * 16, 16)]
                wc[o] = winners_v[pl.ds(j * _DCH + t * 16, 16)]

        npairs = (n + 2 * _DCH - 1) // (2 * _DCH)
        npairs = jnp.maximum(npairs, 1)
        stage(jnp.int32(0), cchunk0, wchunk0)
        pltpu.async_copy(src_hbm.at[wchunk0], rowbuf0, gsem)

        def p4(jj, _):
            for b in (0, 1):
                j = 2 * jj + b
                cc, wc, rb = cbufs[b]
                cc1, wc1, rb1 = cbufs[1 - b]
                stage(j + 1, cc1, wc1)
                pltpu.make_async_copy(src_hbm.at[wc], rb, gsem).wait()
                pltpu.async_copy(src_hbm.at[wc1], rb1, gsem)
                pltpu.async_copy(rb, canvas_hbm.at[cc], ssem).wait()
            return 0
        lax.fori_loop(0, npairs, p4, 0)
        pltpu.make_async_copy(src_hbm.at[wchunk0], rowbuf0, gsem).wait()

        pltpu.sync_copy(map_v, occ_hbm.at[pl.ds(base, _ROWN)])

    return k(flat, src)


def _pass_b(canvas, occ):
    def body(cv_ref, occ_ref, o1_ref, o2_ref):
        b = pl.program_id(0)
        rowmask = lax.broadcasted_iota(jnp.int32, (_B, 1), 0) == b
        occv = jnp.sum(jnp.where(rowmask, occ_ref[...], 0), axis=0)
        v = cv_ref[...]
        occm = (occv > 0)[None, :]
        t = jnp.transpose(v)
        t1 = jnp.where(occm, t[:_CF], 0.0)
        t2 = jnp.where(occm, t[_CF:_CF + 3], 0.0)
        for yy in range(_YB):
            lo, hi = yy * _NX, (yy + 1) * _NX
            o1_ref[0, :, yy, :] = t1[:, lo:hi]
            o2_ref[0, :, yy, :] = t2[:, lo:hi]

    return pl.pallas_call(
        body,
        grid=(_B, _NYB),
        in_specs=[
            pl.BlockSpec((_SBLK, _CW), lambda b, y: (b * _NYB + y, 0)),
            pl.BlockSpec((_B, _SBLK), lambda b, y: (0, y)),
        ],
        out_specs=[
            pl.BlockSpec((1, _CF, _YB, _NX), lambda b, y: (b, 0, y, 0)),
            pl.BlockSpec((1, 3, _YB, _NX), lambda b, y: (b, 0, y, 0)),
        ],
        out_shape=[
            jax.ShapeDtypeStruct((_B, _CF, _NY, _NX), jnp.float32),
            jax.ShapeDtypeStruct((_B, 3, _NY, _NX), jnp.float32),
        ],
    )(canvas, occ)


def kernel(add_features_to_map, pillar_features, voxel_coords):
    p = pillar_features.shape[0]
    vc = voxel_coords.astype(jnp.int32)
    flat = vc[:, 0] * _GRID + vc[:, 1] + vc[:, 2] * _NX + vc[:, 3]
    src = jnp.concatenate(
        [pillar_features, add_features_to_map,
         jnp.zeros((p, _CW - _CF - 3), jnp.float32)], axis=1)

    canvas, occ = _pass_a(flat, src)
    return _pass_b(canvas, occ.reshape(_B, _GRID))


# speedup vs baseline: 1.3244x; 1.2232x over previous
import functools

import jax
import jax.numpy as jnp
from jax import lax
from jax.experimental import pallas as pl
from jax.experimental.pallas import tpu as pltpu, tpu_sc as plsc

_NX, _NY, _NZ = 432, 496, 1
_B = 4
_CF = 64
_GRID = _NZ * _NX * _NY
_RTOT = _B * _GRID
_YB = 16
_SBLK = _YB * _NX
_NYB = _NY // _YB
_CW = 128
_CAN_ROWS = _RTOT + _SBLK
_NW = 32
_ROWN = _RTOT // _NW
_CAP = ((_ROWN + 127) // 128) * 128 + 128
_DCH = 128


def _pass_a(flat, src):
    p = flat.shape[0]
    scch = 8000
    assert p % scch == 0 and p % 16 == 0
    nch_scan = p // scch
    mesh = plsc.VectorSubcoreMesh(core_axis_name="c", subcore_axis_name="s")

    @functools.partial(
        pl.kernel,
        out_type=[
            jax.ShapeDtypeStruct((_CAN_ROWS, _CW), jnp.float32),
            jax.ShapeDtypeStruct((_RTOT,), jnp.int32),
        ],
        mesh=mesh,
        compiler_params=pltpu.CompilerParams(needs_layout_passes=False),
        scratch_types=[
            pltpu.VMEM((_ROWN,), jnp.int32),
            pltpu.VMEM((_CAP,), jnp.int32),
            pltpu.VMEM((_CAP,), jnp.int32),
            pltpu.VMEM((scch,), jnp.int32),
            pltpu.VMEM((scch,), jnp.int32),
            pltpu.VMEM((_DCH,), jnp.int32),
            pltpu.VMEM((_DCH,), jnp.int32),
            pltpu.VMEM((_DCH,), jnp.int32),
            pltpu.VMEM((_DCH,), jnp.int32),
            pltpu.VMEM((_DCH, _CW), jnp.float32),
            pltpu.VMEM((_DCH, _CW), jnp.float32),
            pltpu.SemaphoreType.DMA,
            pltpu.SemaphoreType.DMA,
        ],
    )
    def k(flat_hbm, src_hbm, canvas_hbm, occ_hbm,
          map_v, cells_v, winners_v, idx0, idx1, cchunk0, wchunk0,
          cchunk1, wchunk1, rowbuf0, rowbuf1, gsem, ssem):
        nc = 2
        wid = lax.axis_index("s") * nc + lax.axis_index("c")
        base = wid * _ROWN
        iota = lax.iota(jnp.int32, 16)
        zeros16 = jnp.zeros((16,), jnp.int32)

        def p0(i, _):
            map_v[pl.ds(i * 16, 16)] = zeros16
            return 0
        lax.fori_loop(0, _ROWN // 16, p0, 0)

        def scan_chunk(buf, c):
            def p2(i, _):
                idx = buf[pl.ds(i * 16, 16)]
                m = (idx >= base) & (idx < base + _ROWN)
                ids = c * scch + i * 16 + 1 + iota
                plsc.store_scatter(map_v, [idx - base], ids, mask=m)
                return 0
            lax.fori_loop(0, scch // 16, p2, 0)

        bufs = [idx0, idx1]
        pltpu.async_copy(flat_hbm.at[pl.ds(0, scch)], bufs[0], gsem)
        for c in range(nch_scan):
            buf = bufs[c % 2]
            pltpu.make_async_copy(
                flat_hbm.at[pl.ds(c * scch, scch)], buf, gsem).wait()
            if c + 1 < nch_scan:
                pltpu.async_copy(flat_hbm.at[pl.ds((c + 1) * scch, scch)],
                                 bufs[(c + 1) % 2], gsem)
            scan_chunk(buf, c)

        def p3(g, n):
            v = map_v[pl.ds(g * 16, 16)]
            m = v > 0
            plsc.store_compressed(cells_v.at[pl.ds(n, 16)],
                                  base + g * 16 + iota, mask=m)
            plsc.store_compressed(winners_v.at[pl.ds(n, 16)], v - 1, mask=m)
            return n + jnp.sum(m.astype(jnp.int32))
        n = lax.fori_loop(0, _ROWN // 16, p3, jnp.int32(0))

        for kk in range(9):
            o = pl.ds(n + kk * 16, 16)
            cells_v[o] = _RTOT + ((kk * 16 + iota) & 127)
            winners_v[o] = zeros16

        def p4(j, _):
            for t in range(_DCH // 16):
                o = pl.ds(t * 16, 16)
                cchunk0[o] = cells_v[pl.ds(j * _DCH + t * 16, 16)]
                wchunk0[o] = winners_v[pl.ds(j * _DCH + t * 16, 16)]
            pltpu.async_copy(src_hbm.at[wchunk0], rowbuf0, gsem).wait()
            pltpu.async_copy(rowbuf0, canvas_hbm.at[cchunk0], ssem).wait()
            return 0
        lax.fori_loop(0, (n + _DCH - 1) // _DCH, p4, 0)

        pltpu.sync_copy(map_v, occ_hbm.at[pl.ds(base, _ROWN)])

    return k(flat, src)


def _pass_b(canvas, occ):
    def body(cv_ref, occ_ref, o1_ref, o2_ref):
        b = pl.program_id(0)
        rowmask = lax.broadcasted_iota(jnp.int32, (_B, 1), 0) == b
        occv = jnp.sum(jnp.where(rowmask, occ_ref[...], 0), axis=0)
        v = cv_ref[...]
        occm = (occv > 0)[None, :]
        t = jnp.transpose(v)
        t1 = jnp.where(occm, t[:_CF], 0.0)
        t2 = jnp.where(occm, t[_CF:_CF + 3], 0.0)
        for yy in range(_YB):
            lo, hi = yy * _NX, (yy + 1) * _NX
            o1_ref[0, :, yy, :] = t1[:, lo:hi]
            o2_ref[0, :, yy, :] = t2[:, lo:hi]

    return pl.pallas_call(
        body,
        grid=(_B, _NYB),
        in_specs=[
            pl.BlockSpec((_SBLK, _CW), lambda b, y: (b * _NYB + y, 0)),
            pl.BlockSpec((_B, _SBLK), lambda b, y: (0, y)),
        ],
        out_specs=[
            pl.BlockSpec((1, _CF, _YB, _NX), lambda b, y: (b, 0, y, 0)),
            pl.BlockSpec((1, 3, _YB, _NX), lambda b, y: (b, 0, y, 0)),
        ],
        out_shape=[
            jax.ShapeDtypeStruct((_B, _CF, _NY, _NX), jnp.float32),
            jax.ShapeDtypeStruct((_B, 3, _NY, _NX), jnp.float32),
        ],
    )(canvas, occ)


def kernel(add_features_to_map, pillar_features, voxel_coords):
    p = pillar_features.shape[0]
    vc = voxel_coords.astype(jnp.int32)
    flat = vc[:, 0] * _GRID + vc[:, 1] + vc[:, 2] * _NX + vc[:, 3]
    src = jnp.concatenate(
        [pillar_features, add_features_to_map,
         jnp.zeros((p, _CW - _CF - 3), jnp.float32)], axis=1)

    canvas, occ = _pass_a(flat, src)
    return _pass_b(canvas, occ.reshape(_B, _GRID))


# speedup vs baseline: 1.3252x; 1.0006x over previous
import functools

import jax
import jax.numpy as jnp
from jax import lax
from jax.experimental import pallas as pl
from jax.experimental.pallas import tpu as pltpu, tpu_sc as plsc

_NX, _NY, _NZ = 432, 496, 1
_B = 4
_CF = 64
_GRID = _NZ * _NX * _NY
_RTOT = _B * _GRID
_YB = 16
_SBLK = _YB * _NX
_NYB = _NY // _YB
_CW = 128
_CAN_ROWS = _RTOT + _SBLK
_NW = 32
_ROWN = _RTOT // _NW
_CAP = ((_ROWN + 127) // 128) * 128 + 128
_DCH = 128


def _pass_a(flat, src):
    p = flat.shape[0]
    scch = 8000
    assert p % scch == 0 and p % 16 == 0
    nch_scan = p // scch
    mesh = plsc.VectorSubcoreMesh(core_axis_name="c", subcore_axis_name="s")

    @functools.partial(
        pl.kernel,
        out_type=[
            jax.ShapeDtypeStruct((_CAN_ROWS, _CW), jnp.float32),
            jax.ShapeDtypeStruct((_RTOT,), jnp.int32),
        ],
        mesh=mesh,
        compiler_params=pltpu.CompilerParams(needs_layout_passes=False),
        scratch_types=[
            pltpu.VMEM((_ROWN,), jnp.int32),
            pltpu.VMEM((_CAP,), jnp.int32),
            pltpu.VMEM((_CAP,), jnp.int32),
            pltpu.VMEM((scch,), jnp.int32),
            pltpu.VMEM((scch,), jnp.int32),
            pltpu.VMEM((_DCH,), jnp.int32),
            pltpu.VMEM((_DCH,), jnp.int32),
            pltpu.VMEM((_DCH, _CW), jnp.float32),
            pltpu.SemaphoreType.DMA,
            pltpu.SemaphoreType.DMA,
        ],
    )
    def k(flat_hbm, src_hbm, canvas_hbm, occ_hbm,
          map_v, cells_v, winners_v, idx0, idx1, cchunk0, wchunk0,
          rowbuf0, gsem, ssem):
        nc = 2
        wid = lax.axis_index("s") * nc + lax.axis_index("c")
        base = wid * _ROWN
        iota = lax.iota(jnp.int32, 16)
        zeros16 = jnp.zeros((16,), jnp.int32)

        def p0(i, _):
            map_v[pl.ds(i * 16, 16)] = zeros16
            return 0
        lax.fori_loop(0, _ROWN // 16, p0, 0)

        def scan_chunk(buf, c):
            def p2(i, _):
                idx = buf[pl.ds(i * 16, 16)]
                m = (idx >= base) & (idx < base + _ROWN)
                ids = c * scch + i * 16 + 1 + iota
                plsc.store_scatter(map_v, [idx - base], ids, mask=m)
                return 0
            lax.fori_loop(0, scch // 16, p2, 0)

        bufs = [idx0, idx1]
        pltpu.async_copy(flat_hbm.at[pl.ds(0, scch)], bufs[0], gsem)
        for c in range(nch_scan):
            buf = bufs[c % 2]
            pltpu.make_async_copy(
                flat_hbm.at[pl.ds(c * scch, scch)], buf, gsem).wait()
            if c + 1 < nch_scan:
                pltpu.async_copy(flat_hbm.at[pl.ds((c + 1) * scch, scch)],
                                 bufs[(c + 1) % 2], gsem)
            scan_chunk(buf, c)

        def p3(g, n):
            v = map_v[pl.ds(g * 16, 16)]
            m = v > 0
            plsc.store_compressed(cells_v.at[pl.ds(n, 16)],
                                  base + g * 16 + iota, mask=m)
            plsc.store_compressed(winners_v.at[pl.ds(n, 16)], v - 1, mask=m)
            return n + jnp.sum(m.astype(jnp.int32))
        n = lax.fori_loop(0, _ROWN // 16, p3, jnp.int32(0))

        for kk in range(9):
            o = pl.ds(n + kk * 16, 16)
            cells_v[o] = _RTOT + ((kk * 16 + iota) & 127)
            winners_v[o] = zeros16

        def p4(j, _):
            for t in range(_DCH // 16):
                o = pl.ds(t * 16, 16)
                cchunk0[o] = cells_v[pl.ds(j * _DCH + t * 16, 16)]
                wchunk0[o] = winners_v[pl.ds(j * _DCH + t * 16, 16)]
            pltpu.async_copy(src_hbm.at[wchunk0], rowbuf0, gsem).wait()
            pltpu.async_copy(rowbuf0, canvas_hbm.at[cchunk0], ssem).wait()
            return 0
        lax.fori_loop(0, (n + _DCH - 1) // _DCH, p4, 0)

        pltpu.sync_copy(map_v, occ_hbm.at[pl.ds(base, _ROWN)])

    return k(flat, src)


def _pass_b(canvas, occ):
    def body(cv_ref, occ_ref, o1_ref, o2_ref):
        b = pl.program_id(0)
        rowmask = lax.broadcasted_iota(jnp.int32, (_B, 1), 0) == b
        occv = jnp.sum(jnp.where(rowmask, occ_ref[...], 0), axis=0)
        v = cv_ref[...]
        occm = (occv > 0)[None, :]
        t = jnp.transpose(v)
        t1 = jnp.where(occm, t[:_CF], 0.0)
        t2 = jnp.where(occm, t[_CF:_CF + 3], 0.0)
        for yy in range(_YB):
            lo, hi = yy * _NX, (yy + 1) * _NX
            o1_ref[0, :, yy, :] = t1[:, lo:hi]
            o2_ref[0, :, yy, :] = t2[:, lo:hi]

    return pl.pallas_call(
        body,
        grid=(_B, _NYB),
        in_specs=[
            pl.BlockSpec((_SBLK, _CW), lambda b, y: (b * _NYB + y, 0)),
            pl.BlockSpec((_B, _SBLK), lambda b, y: (0, y)),
        ],
        out_specs=[
            pl.BlockSpec((1, _CF, _YB, _NX), lambda b, y: (b, 0, y, 0)),
            pl.BlockSpec((1, 3, _YB, _NX), lambda b, y: (b, 0, y, 0)),
        ],
        out_shape=[
            jax.ShapeDtypeStruct((_B, _CF, _NY, _NX), jnp.float32),
            jax.ShapeDtypeStruct((_B, 3, _NY, _NX), jnp.float32),
        ],
    )(canvas, occ)


def kernel(add_features_to_map, pillar_features, voxel_coords):
    p = pillar_features.shape[0]
    vc = voxel_coords.astype(jnp.int32)
    flat = vc[:, 0] * _GRID + vc[:, 1] + vc[:, 2] * _NX + vc[:, 3]
    src = jnp.concatenate(
        [pillar_features, add_features_to_map,
         jnp.zeros((p, _CW - _CF - 3), jnp.float32)], axis=1)

    canvas, occ = _pass_a(flat, src)
    return _pass_b(canvas, occ.reshape(_B, _GRID))
